# baseline (speedup 1.0000x reference)
import functools
import jax
import jax.numpy as jnp
from jax import lax
from jax.experimental import pallas as pl
from jax.experimental.pallas import tpu as pltpu
from jax.experimental.pallas import tpu_sc as plsc

_N = 10000
_NP = 10016
_D = 256
_HALF = 128
_CH = 128
_K = 4
_TILES = 16
_ROWS_PER_TILE = _NP // _TILES
_DW = 16

import numpy as _np
_PERM128 = _np.concatenate(
    [_np.concatenate([_np.arange(32 * t, 32 * t + 32, 2),
                      _np.arange(32 * t + 1, 32 * t + 32, 2)])
     for t in range(128 // 32)])
_PERM256 = _np.concatenate([_PERM128, _PERM128 + 128])


def _sc_segment_sum(xr2, srcs2, dsts, zrows, zdeg, onesrow, nchunks):
    mesh = plsc.VectorSubcoreMesh(core_axis_name="c", subcore_axis_name="s")
    nsup = nchunks // _K

    @functools.partial(
        pl.kernel,
        out_type=(jax.ShapeDtypeStruct((2, _NP, _HALF), jnp.float32),
                  jax.ShapeDtypeStruct((_NP, _DW), jnp.float32)),
        mesh=mesh,
        compiler_params=pltpu.CompilerParams(use_tc_tiling_on_sc=False,
                                             needs_layout_passes=False),
        scratch_types=[
            pltpu.VMEM((2, _K, _CH), jnp.int32),
            pltpu.VMEM((2, _K, _CH), jnp.int32),
            pltpu.VMEM((_CH, _HALF), jnp.bfloat16),
            pltpu.VMEM((_CH, _HALF), jnp.bfloat16),
            pltpu.VMEM((_CH, _HALF), jnp.float32),
            pltpu.VMEM((_CH, _DW), jnp.float32),
            pltpu.SemaphoreType.DMA,
            pltpu.SemaphoreType.DMA,
            pltpu.VMEM_SHARED((_NP, _HALF), jnp.float32),
            pltpu.VMEM_SHARED((_NP, _DW), jnp.float32),
        ],
    )
    def k(xr2_hbm, src_hbm, dst_hbm, z_hbm, zd_hbm, ones_hbm,
          agg_out, deg_out,
          src_sup, dst_sup, rows0, rows1, rows_f32, ones_v, sem0, sem1,
          agg_sh, deg_sh):
        cid = lax.axis_index("c")
        sid = lax.axis_index("s")
        row0 = sid * _ROWS_PER_TILE
        rows_v = (rows0, rows1)
        sems = (sem0, sem1)
        crow0 = sid * nchunks

        def load_sup(s, sb):
            pltpu.sync_copy(src_hbm.at[cid, pl.ds(crow0 + s * _K, _K)],
                            src_sup.at[sb])
            pltpu.sync_copy(dst_hbm.at[pl.ds(crow0 + s * _K, _K)],
                            dst_sup.at[sb])

        def fire(sb, j, b):
            pltpu.async_copy(xr2_hbm.at[src_sup.at[sb, j]],
                             rows_v[b], sems[b])

        def wait(sb, j, b):
            pltpu.make_async_copy(xr2_hbm.at[src_sup.at[sb, j]],
                                  rows_v[b], sems[b]).wait()

        pltpu.sync_copy(ones_hbm, ones_v)
        pltpu.sync_copy(z_hbm, agg_sh.at[pl.ds(row0, _ROWS_PER_TILE)])

        @pl.when(cid == 0)
        def _():
            pltpu.sync_copy(zd_hbm, deg_sh.at[pl.ds(row0, _ROWS_PER_TILE)])

        plsc.subcore_barrier()

        load_sup(0, 0)
        fire(0, 0, 0)

        def body(i, carry):
            for sp in range(2):
                s = 2 * i + sp
                sb = sp
                load_sup(s + 1, 1 - sb)
                for j in range(_K):
                    b = j % 2
                    jn, sbn = (j + 1, sb) if j + 1 < _K else (0, 1 - sb)
                    fire(sbn, jn, 1 - b)
                    wait(sb, j, b)

                    def conv(r, carry2, _rb=rows_v[b]):
                        for t in range(_HALF // 32):
                            w = plsc.bitcast(_rb[r, pl.ds(32 * t, 32)],
                                             jnp.int32)
                            ev = plsc.bitcast(lax.shift_left(w, 16),
                                              jnp.float32)
                            od = plsc.bitcast(
                                lax.bitwise_and(w, jnp.int32(-65536)),
                                jnp.float32)
                            rows_f32[r, pl.ds(32 * t, 16)] = ev
                            rows_f32[r, pl.ds(32 * t + 16, 16)] = od
                        return carry2

                    lax.fori_loop(0, _CH, conv, 0)
                    pltpu.sync_copy(rows_f32,
                                    agg_sh.at[dst_sup.at[sb, j]], add=True)

                    @pl.when(cid == 0)
                    def _():
                        pltpu.sync_copy(ones_v,
                                        deg_sh.at[dst_sup.at[sb, j]],
                                        add=True)
            return carry

        lax.fori_loop(0, nsup // 2, body, 0)
        wait(0, 0, 0)

        plsc.subcore_barrier()
        pltpu.sync_copy(agg_sh.at[pl.ds(row0, _ROWS_PER_TILE)],
                        agg_out.at[cid, pl.ds(row0, _ROWS_PER_TILE)])

        @pl.when(cid == 0)
        def _():
            pltpu.sync_copy(deg_sh.at[pl.ds(row0, _ROWS_PER_TILE)],
                            deg_out.at[pl.ds(row0, _ROWS_PER_TILE)])

    return k(xr2, srcs2, dsts, zrows, zdeg, onesrow)


def _tc_body(a0, a1, dg, x, wl, bl, wr, wd1, bd1, wd2, bd2, o):
    deg = jnp.clip(dg[:, :1], 1.0, None)
    agg = jnp.concatenate([a0[...], a1[...]], axis=1) / deg
    h = jnp.dot(agg, wl[...], preferred_element_type=jnp.float32)
    h += jnp.dot(x[...], wr[...], preferred_element_type=jnp.float32)
    h = jnp.maximum(h + bl[...], 0.0)
    hd = jnp.dot(h, wd1[...], preferred_element_type=jnp.float32)
    hd = jnp.maximum(hd + bd1[...], 0.0)
    out = jnp.dot(hd, wd2[...], preferred_element_type=jnp.float32)
    o[...] = out + bd2[...]


def _tc_dense(a0, a1, dg, x, W_l, b_l, W_r, W_d1, b_d1, W_d2, b_d2):
    n = x.shape[0]
    B = 512
    grid = (pl.cdiv(_NP, B),)

    def row_blk(cols):
        return pl.BlockSpec((B, cols), lambda i: (i, 0))

    def full(shape):
        return pl.BlockSpec(shape, lambda i: tuple(0 for _ in shape))

    return pl.pallas_call(
        _tc_body,
        grid=grid,
        in_specs=[
            row_blk(_HALF), row_blk(_HALF), row_blk(_DW), row_blk(_D),
            full(W_l.shape), full(b_l.shape), full(W_r.shape),
            full(W_d1.shape), full(b_d1.shape),
            full(W_d2.shape), full(b_d2.shape),
        ],
        out_specs=row_blk(_D),
        out_shape=jax.ShapeDtypeStruct((n, _D), jnp.float32),
    )(a0, a1, dg, x, W_l, b_l, W_r, W_d1, b_d1, W_d2, b_d2)


def kernel(x, edge_index, W_l, b_l, W_r, W_d1, b_d1, W_d2, b_d2):
    n = x.shape[0]
    e = edge_index.shape[1]
    f32 = jnp.float32

    src = edge_index[0].astype(jnp.int32)
    dst = edge_index[1].astype(jnp.int32)

    xr2 = x.astype(jnp.bfloat16).reshape(2 * n, _HALF)

    step = _TILES * _CH
    kk = 2 * _K
    nchunks = ((e + step - 1) // step + kk - 1) // kk * kk
    ep = nchunks * step
    nr = ep // _CH + _K
    srcp = jnp.concatenate(
        [src, jnp.zeros((nr * _CH - e,), jnp.int32)]).reshape(nr, _CH)
    dstp = jnp.concatenate(
        [dst, jnp.full((nr * _CH - e,), n, jnp.int32)]).reshape(nr, _CH)
    src2 = 2 * srcp
    srcs2 = jnp.stack([src2, src2 + 1])
    zrows = jnp.zeros((_ROWS_PER_TILE, _HALF), f32)
    zdeg = jnp.zeros((_ROWS_PER_TILE, _DW), f32)
    onesrow = jnp.ones((_CH, _DW), f32)

    agg2, deg = _sc_segment_sum(xr2, srcs2, dstp, zrows, zdeg, onesrow,
                                nchunks)

    b_l2 = b_l.reshape(1, -1)
    b_d12 = b_d1.reshape(1, -1)
    b_d22 = b_d2.reshape(1, -1)
    return _tc_dense(agg2[0], agg2[1], deg, x, W_l[_PERM256], b_l2, W_r,
                     W_d1, b_d12, W_d2, b_d22)


# speedup vs baseline: 1.0947x; 1.0947x over previous
import functools
import jax
import jax.numpy as jnp
from jax import lax
from jax.experimental import pallas as pl
from jax.experimental.pallas import tpu as pltpu
from jax.experimental.pallas import tpu_sc as plsc

_N = 10000
_NP = 10016
_D = 256
_HALF = 128
_CH = 128
_K = 4
_TILES = 16
_ROWS_PER_TILE = _NP // _TILES
_DW = 16


def _sc_segment_sum(xr2, srcs2, dsts, zrows, zdeg, onesrow, nchunks):
    mesh = plsc.VectorSubcoreMesh(core_axis_name="c", subcore_axis_name="s")
    nsup = nchunks // _K

    @functools.partial(
        pl.kernel,
        out_type=(jax.ShapeDtypeStruct((2, _NP, _HALF), jnp.float32),
                  jax.ShapeDtypeStruct((_NP, _DW), jnp.float32)),
        mesh=mesh,
        compiler_params=pltpu.CompilerParams(use_tc_tiling_on_sc=False),
        scratch_types=[
            pltpu.VMEM((2, _K, _CH), jnp.int32),
            pltpu.VMEM((2, _K, _CH), jnp.int32),
            pltpu.VMEM((_CH, _HALF), jnp.float32),
            pltpu.VMEM((_CH, _HALF), jnp.float32),
            pltpu.VMEM((_CH, _DW), jnp.float32),
            pltpu.SemaphoreType.DMA,
            pltpu.SemaphoreType.DMA,
            pltpu.VMEM_SHARED((_NP, _HALF), jnp.float32),
            pltpu.VMEM_SHARED((_NP, _DW), jnp.float32),
        ],
    )
    def k(xr2_hbm, src_hbm, dst_hbm, z_hbm, zd_hbm, ones_hbm,
          agg_out, deg_out,
          src_sup, dst_sup, rows0, rows1, ones_v, sem0, sem1,
          agg_sh, deg_sh):
        cid = lax.axis_index("c")
        sid = lax.axis_index("s")
        row0 = sid * _ROWS_PER_TILE
        rows_v = (rows0, rows1)
        sems = (sem0, sem1)
        crow0 = sid * nchunks

        def load_sup(s, sb):
            pltpu.sync_copy(src_hbm.at[cid, pl.ds(crow0 + s * _K, _K)],
                            src_sup.at[sb])
            pltpu.sync_copy(dst_hbm.at[pl.ds(crow0 + s * _K, _K)],
                            dst_sup.at[sb])

        def fire(sb, j, b):
            pltpu.async_copy(xr2_hbm.at[src_sup.at[sb, j]],
                             rows_v[b], sems[b])

        def wait(sb, j, b):
            pltpu.make_async_copy(xr2_hbm.at[src_sup.at[sb, j]],
                                  rows_v[b], sems[b]).wait()

        pltpu.sync_copy(ones_hbm, ones_v)
        pltpu.sync_copy(z_hbm, agg_sh.at[pl.ds(row0, _ROWS_PER_TILE)])

        @pl.when(cid == 0)
        def _():
            pltpu.sync_copy(zd_hbm, deg_sh.at[pl.ds(row0, _ROWS_PER_TILE)])

        plsc.subcore_barrier()

        load_sup(0, 0)
        fire(0, 0, 0)

        def body(i, carry):
            for sp in range(2):
                s = 2 * i + sp
                sb = sp
                load_sup(s + 1, 1 - sb)
                for j in range(_K):
                    b = j % 2
                    jn, sbn = (j + 1, sb) if j + 1 < _K else (0, 1 - sb)
                    fire(sbn, jn, 1 - b)
                    wait(sb, j, b)
                    pltpu.sync_copy(rows_v[b],
                                    agg_sh.at[dst_sup.at[sb, j]], add=True)

                    @pl.when(cid == 0)
                    def _():
                        pltpu.sync_copy(ones_v,
                                        deg_sh.at[dst_sup.at[sb, j]],
                                        add=True)
            return carry

        lax.fori_loop(0, nsup // 2, body, 0)
        wait(0, 0, 0)

        plsc.subcore_barrier()
        pltpu.sync_copy(agg_sh.at[pl.ds(row0, _ROWS_PER_TILE)],
                        agg_out.at[cid, pl.ds(row0, _ROWS_PER_TILE)])

        @pl.when(cid == 0)
        def _():
            pltpu.sync_copy(deg_sh.at[pl.ds(row0, _ROWS_PER_TILE)],
                            deg_out.at[pl.ds(row0, _ROWS_PER_TILE)])

    return k(xr2, srcs2, dsts, zrows, zdeg, onesrow)


def _tc_xwr_body(x, wr, bl, o):
    o[...] = jnp.dot(x[...], wr[...],
                     preferred_element_type=jnp.float32) + bl[...]


def _tc_xwr(x, W_r, b_l):
    n = x.shape[0]
    B = 512
    return pl.pallas_call(
        _tc_xwr_body,
        grid=(pl.cdiv(n, B),),
        in_specs=[
            pl.BlockSpec((B, _D), lambda i: (i, 0)),
            pl.BlockSpec(W_r.shape, lambda i: (0, 0)),
            pl.BlockSpec(b_l.shape, lambda i: (0, 0)),
        ],
        out_specs=pl.BlockSpec((B, 512), lambda i: (i, 0)),
        out_shape=jax.ShapeDtypeStruct((n, 512), jnp.float32),
    )(x, W_r, b_l)


def _tc_body(a0, a1, dg, xwr, wl, wd1, bd1, wd2, bd2, o):
    deg = jnp.clip(dg[:, :1], 1.0, None)
    agg = jnp.concatenate([a0[...], a1[...]], axis=1) / deg
    h = jnp.dot(agg, wl[...], preferred_element_type=jnp.float32)
    h += xwr[...]
    h = jnp.maximum(h, 0.0)
    hd = jnp.dot(h, wd1[...], preferred_element_type=jnp.float32)
    hd = jnp.maximum(hd + bd1[...], 0.0)
    out = jnp.dot(hd, wd2[...], preferred_element_type=jnp.float32)
    o[...] = out + bd2[...]


def _tc_dense(a0, a1, dg, xwr, W_l, W_d1, b_d1, W_d2, b_d2):
    n = xwr.shape[0]
    B = 512
    grid = (pl.cdiv(n, B),)

    def row_blk(cols):
        return pl.BlockSpec((B, cols), lambda i: (i, 0))

    def full(shape):
        return pl.BlockSpec(shape, lambda i: tuple(0 for _ in shape))

    return pl.pallas_call(
        _tc_body,
        grid=grid,
        in_specs=[
            row_blk(_HALF), row_blk(_HALF), row_blk(_DW), row_blk(512),
            full(W_l.shape),
            full(W_d1.shape), full(b_d1.shape),
            full(W_d2.shape), full(b_d2.shape),
        ],
        out_specs=row_blk(_D),
        out_shape=jax.ShapeDtypeStruct((n, _D), jnp.float32),
    )(a0, a1, dg, xwr, W_l, W_d1, b_d1, W_d2, b_d2)


def kernel(x, edge_index, W_l, b_l, W_r, W_d1, b_d1, W_d2, b_d2):
    n = x.shape[0]
    e = edge_index.shape[1]
    f32 = jnp.float32

    src = edge_index[0].astype(jnp.int32)
    dst = edge_index[1].astype(jnp.int32)

    xr2 = x.reshape(2 * n, _HALF)

    step = _TILES * _CH
    kk = 2 * _K
    nchunks = ((e + step - 1) // step + kk - 1) // kk * kk
    ep = nchunks * step
    nr = ep // _CH + _K
    srcp = jnp.concatenate(
        [src, jnp.zeros((nr * _CH - e,), jnp.int32)]).reshape(nr, _CH)
    dstp = jnp.concatenate(
        [dst, jnp.full((nr * _CH - e,), n, jnp.int32)]).reshape(nr, _CH)
    src2 = 2 * srcp
    srcs2 = jnp.stack([src2, src2 + 1])
    zrows = jnp.zeros((_ROWS_PER_TILE, _HALF), f32)
    zdeg = jnp.zeros((_ROWS_PER_TILE, _DW), f32)
    onesrow = jnp.ones((_CH, _DW), f32)

    xwr = _tc_xwr(x, W_r, b_l.reshape(1, -1))

    agg2, deg = _sc_segment_sum(xr2, srcs2, dstp, zrows, zdeg, onesrow,
                                nchunks)

    b_d12 = b_d1.reshape(1, -1)
    b_d22 = b_d2.reshape(1, -1)
    return _tc_dense(agg2[0], agg2[1], deg, xwr, W_l,
                     W_d1, b_d12, W_d2, b_d22)


# speedup vs baseline: 1.1199x; 1.0230x over previous
import functools
import jax
import jax.numpy as jnp
from jax import lax
from jax.experimental import pallas as pl
from jax.experimental.pallas import tpu as pltpu
from jax.experimental.pallas import tpu_sc as plsc

_N = 10000
_NP = 10016
_D = 256
_HALF = 128
_CH = 128
_K = 4
_TILES = 16
_ROWS_PER_TILE = _NP // _TILES
_DW = 16


def _sc_segment_sum(xr2, srcs2, dsts, zrows, zdeg, onesrow, nchunks):
    mesh = plsc.VectorSubcoreMesh(core_axis_name="c", subcore_axis_name="s")
    nsup = nchunks // _K

    @functools.partial(
        pl.kernel,
        out_type=(jax.ShapeDtypeStruct((2, _NP, _HALF), jnp.float32),
                  jax.ShapeDtypeStruct((_NP, _DW), jnp.float32)),
        mesh=mesh,
        compiler_params=pltpu.CompilerParams(use_tc_tiling_on_sc=False),
        scratch_types=[
            pltpu.VMEM((2, _K, _CH), jnp.int32),
            pltpu.VMEM((2, _K, _CH), jnp.int32),
            pltpu.VMEM((_CH, _HALF), jnp.float32),
            pltpu.VMEM((_CH, _HALF), jnp.float32),
            pltpu.VMEM((_CH, _DW), jnp.float32),
            pltpu.SemaphoreType.DMA,
            pltpu.SemaphoreType.DMA,
            pltpu.VMEM_SHARED((_NP, _HALF), jnp.float32),
            pltpu.VMEM_SHARED((_NP, _DW), jnp.float32),
        ],
    )
    def k(xr2_hbm, src_hbm, dst_hbm, z_hbm, zd_hbm, ones_hbm,
          agg_out, deg_out,
          src_sup, dst_sup, rows0, rows1, ones_v, sem0, sem1,
          agg_sh, deg_sh):
        cid = lax.axis_index("c")
        sid = lax.axis_index("s")
        row0 = sid * _ROWS_PER_TILE
        rows_v = (rows0, rows1)
        sems = (sem0, sem1)
        crow0 = sid * nchunks

        def load_sup(s, sb):
            pltpu.sync_copy(src_hbm.at[cid, pl.ds(crow0 + s * _K, _K)],
                            src_sup.at[sb])
            pltpu.sync_copy(dst_hbm.at[pl.ds(crow0 + s * _K, _K)],
                            dst_sup.at[sb])

        def fire(sb, j, b):
            pltpu.async_copy(xr2_hbm.at[src_sup.at[sb, j]],
                             rows_v[b], sems[b])

        def wait(sb, j, b):
            pltpu.make_async_copy(xr2_hbm.at[src_sup.at[sb, j]],
                                  rows_v[b], sems[b]).wait()

        pltpu.sync_copy(ones_hbm, ones_v)
        pltpu.sync_copy(z_hbm, agg_sh.at[pl.ds(row0, _ROWS_PER_TILE)])

        @pl.when(cid == 0)
        def _():
            pltpu.sync_copy(zd_hbm, deg_sh.at[pl.ds(row0, _ROWS_PER_TILE)])

        plsc.subcore_barrier()

        load_sup(0, 0)
        fire(0, 0, 0)

        def body(i, carry):
            for sp in range(2):
                s = 2 * i + sp
                sb = sp
                load_sup(s + 1, 1 - sb)
                for j in range(_K):
                    b = j % 2
                    jn, sbn = (j + 1, sb) if j + 1 < _K else (0, 1 - sb)
                    fire(sbn, jn, 1 - b)
                    wait(sb, j, b)
                    pltpu.sync_copy(rows_v[b],
                                    agg_sh.at[dst_sup.at[sb, j]], add=True)

                    @pl.when(cid == 0)
                    def _():
                        pltpu.sync_copy(ones_v,
                                        deg_sh.at[dst_sup.at[sb, j]],
                                        add=True)
            return carry

        lax.fori_loop(0, nsup // 2, body, 0)
        wait(0, 0, 0)

        plsc.subcore_barrier()
        pltpu.sync_copy(agg_sh.at[pl.ds(row0, _ROWS_PER_TILE)],
                        agg_out.at[cid, pl.ds(row0, _ROWS_PER_TILE)])

        @pl.when(cid == 0)
        def _():
            pltpu.sync_copy(deg_sh.at[pl.ds(row0, _ROWS_PER_TILE)],
                            deg_out.at[pl.ds(row0, _ROWS_PER_TILE)])

    return k(xr2, srcs2, dsts, zrows, zdeg, onesrow)


def _tc_xwr_body(x, wr, bl, o):
    o[...] = jnp.dot(x[...], wr[...],
                     preferred_element_type=jnp.float32) + bl[...]


def _tc_xwr(x, W_r, b_l):
    n = x.shape[0]
    B = 512
    return pl.pallas_call(
        _tc_xwr_body,
        grid=(pl.cdiv(n, B),),
        in_specs=[
            pl.BlockSpec((B, _D), lambda i: (i, 0)),
            pl.BlockSpec(W_r.shape, lambda i: (0, 0)),
            pl.BlockSpec(b_l.shape, lambda i: (0, 0)),
        ],
        out_specs=pl.BlockSpec((B, 512), lambda i: (i, 0)),
        out_shape=jax.ShapeDtypeStruct((n, 512), jnp.float32),
    )(x, W_r, b_l)


def _tc_body(a0, a1, dg, xwr, wl, wd1, bd1, wd2, bd2, o):
    deg = jnp.clip(dg[:, :1], 1.0, None)
    agg = jnp.concatenate([a0[...], a1[...]], axis=1) / deg
    h = jnp.dot(agg, wl[...], preferred_element_type=jnp.float32)
    h += xwr[...]
    h = jnp.maximum(h, 0.0)
    hd = jnp.dot(h, wd1[...], preferred_element_type=jnp.float32)
    hd = jnp.maximum(hd + bd1[...], 0.0)
    out = jnp.dot(hd, wd2[...], preferred_element_type=jnp.float32)
    o[...] = out + bd2[...]


def _tc_dense(a0, a1, dg, xwr, W_l, W_d1, b_d1, W_d2, b_d2):
    n = xwr.shape[0]
    B = 1024
    grid = (pl.cdiv(n, B),)

    def row_blk(cols):
        return pl.BlockSpec((B, cols), lambda i: (i, 0))

    def full(shape):
        return pl.BlockSpec(shape, lambda i: tuple(0 for _ in shape))

    return pl.pallas_call(
        _tc_body,
        grid=grid,
        in_specs=[
            row_blk(_HALF), row_blk(_HALF), row_blk(_DW), row_blk(512),
            full(W_l.shape),
            full(W_d1.shape), full(b_d1.shape),
            full(W_d2.shape), full(b_d2.shape),
        ],
        out_specs=row_blk(_D),
        out_shape=jax.ShapeDtypeStruct((n, _D), jnp.float32),
    )(a0, a1, dg, xwr, W_l, W_d1, b_d1, W_d2, b_d2)


def kernel(x, edge_index, W_l, b_l, W_r, W_d1, b_d1, W_d2, b_d2):
    n = x.shape[0]
    e = edge_index.shape[1]
    f32 = jnp.float32

    src = edge_index[0].astype(jnp.int32)
    dst = edge_index[1].astype(jnp.int32)

    xr2 = x.reshape(2 * n, _HALF)

    step = _TILES * _CH
    kk = 2 * _K
    nchunks = ((e + step - 1) // step + kk - 1) // kk * kk
    ep = nchunks * step
    nr = ep // _CH + _K
    srcp = jnp.concatenate(
        [src, jnp.zeros((nr * _CH - e,), jnp.int32)]).reshape(nr, _CH)
    dstp = jnp.concatenate(
        [dst, jnp.full((nr * _CH - e,), n, jnp.int32)]).reshape(nr, _CH)
    src2 = 2 * srcp
    srcs2 = jnp.stack([src2, src2 + 1])
    zrows = jnp.zeros((_ROWS_PER_TILE, _HALF), f32)
    zdeg = jnp.zeros((_ROWS_PER_TILE, _DW), f32)
    onesrow = jnp.ones((_CH, _DW), f32)

    xwr = _tc_xwr(x, W_r, b_l.reshape(1, -1))

    agg2, deg = _sc_segment_sum(xr2, srcs2, dstp, zrows, zdeg, onesrow,
                                nchunks)

    b_d12 = b_d1.reshape(1, -1)
    b_d22 = b_d2.reshape(1, -1)
    return _tc_dense(agg2[0], agg2[1], deg, xwr, W_l,
                     W_d1, b_d12, W_d2, b_d22)
